# Initial kernel scaffold; baseline (speedup 1.0000x reference)
import jax
import jax.numpy as jnp
from jax.experimental import pallas as pl


def kernel(x, edge_index, W1_self, W1_neigh, b1, memory, Wq_m, Wk_in, Wv_in, Wq_t, Wk_m, Wv_m, Wo, bo, g_mem, b_mem, g_attn, b_attn, fc1, bfc1, fc2, bfc2, g_ffn, b_ffn, g_out, b_out, W2_self, W2_neigh, b2):
    raise NotImplementedError("write your pallas kernel here")


# speedup vs baseline: 5.8971x; 5.8971x over previous
import functools

import jax
import jax.numpy as jnp
from jax import lax
from jax.experimental import pallas as pl
from jax.experimental.pallas import tpu as pltpu
from jax.experimental.pallas import tpu_sc as plsc

_N = 10000
_E = 160000
_D_IN = 256
_D_H = 128
_D_OUT = 64
_HEADS = 4
_HD = _D_H // _HEADS
_NPAD = 10240
_BLK = 256
_GRID = _NPAD // _BLK

_NCORES = 2
_NSUB = 16
_NTILES = _NCORES * _NSUB
_EPT = _E // _NTILES
_CHUNK = 128
_NFULL = _EPT // _CHUNK
_TAIL = _EPT - _NFULL * _CHUNK
_RPT = _NPAD // _NSUB


@functools.lru_cache(maxsize=None)
def _make_edge_agg(with_deg):
  mesh = plsc.VectorSubcoreMesh(core_axis_name="c", subcore_axis_name="s")

  psum_type = jax.ShapeDtypeStruct((_NCORES, _NPAD, _D_H), jnp.float32)
  out_type = [psum_type] if with_deg else psum_type
  scratch = [
      pltpu.VMEM((_CHUNK,), jnp.int32),
      pltpu.VMEM((_CHUNK,), jnp.int32),
      pltpu.VMEM((_TAIL,), jnp.int32),
      pltpu.VMEM((_TAIL,), jnp.int32),
      pltpu.VMEM((16,), jnp.int32),
      pltpu.VMEM((_CHUNK, _D_H), jnp.float32),
      pltpu.VMEM((_TAIL, _D_H), jnp.float32),
      pltpu.VMEM_SHARED((_NPAD, _D_H), jnp.float32),
      pltpu.SemaphoreType.DMA,
  ]
  if with_deg:
    out_type.append(jax.ShapeDtypeStruct((_NTILES, _NPAD), jnp.float32))
    scratch.insert(-1, pltpu.VMEM((_NPAD,), jnp.float32))

  @functools.partial(
      pl.kernel, mesh=mesh, out_type=out_type, scratch_types=scratch,
      compiler_params=pltpu.CompilerParams(needs_layout_passes=False))
  def agg(table_hbm, src_hbm, dst_hbm, zeros_hbm, *refs):
    if with_deg:
      (out_hbm, outd_hbm, sidx, didx, sidx_t, didx_t, didx16,
       rows, rows_t, acc, deg, sem) = refs
    else:
      (out_hbm, sidx, didx, sidx_t, didx_t, didx16,
       rows, rows_t, acc, sem) = refs
      deg = None
    c = lax.axis_index("c")
    s = lax.axis_index("s")
    gid = c * _NSUB + s

    ones16 = jnp.ones((16,), jnp.float32)
    zero16 = jnp.zeros((16,), jnp.float32)

    pltpu.sync_copy(zeros_hbm, rows)
    for j in range(_RPT // _CHUNK):
      r0 = pl.multiple_of(s * _RPT + j * _CHUNK, 8)
      pltpu.sync_copy(rows, acc.at[pl.ds(r0, _CHUNK)])
    if with_deg:
      def zbody(i, carry):
        deg[pl.ds(pl.multiple_of(i * 16, 16), 16)] = zero16
        return carry
      lax.fori_loop(0, _NPAD // 16, zbody, 0)
    plsc.subcore_barrier()

    ebase = gid * _EPT

    def body(i, carry):
      b = pl.multiple_of(ebase + i * _CHUNK, 8)
      pltpu.sync_copy(src_hbm.at[pl.ds(b, _CHUNK)], sidx)
      pltpu.sync_copy(dst_hbm.at[pl.ds(b, _CHUNK)], didx)
      gath = pltpu.async_copy(table_hbm.at[sidx], rows, sem)
      if with_deg:
        for j in range(_CHUNK // 16):
          idx16 = didx[pl.ds(j * 16, 16)]
          plsc.addupdate_scatter(deg, [idx16], ones16)
      gath.wait()
      pltpu.sync_copy(rows, acc.at[didx], add=True)
      return carry

    lax.fori_loop(0, _NFULL, body, 0)

    bt = pl.multiple_of(ebase + _NFULL * _CHUNK, 8)
    pltpu.sync_copy(src_hbm.at[pl.ds(bt, _TAIL)], sidx_t)
    pltpu.sync_copy(dst_hbm.at[pl.ds(bt, _TAIL)], didx_t)
    gath = pltpu.async_copy(table_hbm.at[sidx_t], rows_t, sem)
    if with_deg:
      b16 = pl.multiple_of(ebase + _EPT - 16, 8)
      pltpu.sync_copy(dst_hbm.at[pl.ds(b16, 16)], didx16)
      lane = lax.broadcasted_iota(jnp.int32, (16,), 0)
      mask = lane >= (16 - _TAIL)
      plsc.addupdate_scatter(deg, [didx16[...]], ones16, mask=mask)
    gath.wait()
    pltpu.sync_copy(rows_t, acc.at[didx_t], add=True)

    plsc.subcore_barrier()

    for j in range(_RPT // _CHUNK):
      r0 = pl.multiple_of(s * _RPT + j * _CHUNK, 8)
      pltpu.sync_copy(acc.at[pl.ds(r0, _CHUNK)], rows)
      pltpu.sync_copy(rows, out_hbm.at[c, pl.ds(r0, _CHUNK)])
    if with_deg:
      pltpu.sync_copy(deg, outd_hbm.at[gid])

  return agg


def _ln(x, g, b, eps=1e-5):
  m = jnp.mean(x, axis=-1, keepdims=True)
  v = jnp.mean((x - m) * (x - m), axis=-1, keepdims=True)
  return (x - m) / jnp.sqrt(v + eps) * g + b


def _dot(a, b):
  return jnp.dot(a, b, preferred_element_type=jnp.float32)


def _dot_t(a, b):
  return lax.dot_general(a, b, (((1,), (1,)), ((), ())),
                         preferred_element_type=jnp.float32)


def _pre_body(x_ref, w1n_ref, w1s_ref, b1_ref, table_ref, xs_ref):
  xb = x_ref[...]
  table_ref[...] = _dot(xb, w1n_ref[...])
  xs_ref[...] = _dot(xb, w1s_ref[...]) + b1_ref[...]


def _mid_body(xs_ref, p_ref, degp_ref, mem_ref, wqm_ref, wkin_ref, wvin_ref,
              gm_ref, bm_ref, h_ref, deg_ref, memnew_ref):
  sums = p_ref[0] + p_ref[1]
  deg = jnp.maximum(jnp.sum(degp_ref[...], axis=0), 1.0)
  h = jnp.maximum(xs_ref[...] + sums / deg[:, None], 0.0)
  h_ref[...] = h
  deg_ref[...] = deg

  qm = _dot(mem_ref[...], wqm_ref[...])
  k = _dot(h, wkin_ref[...])
  v = _dot(h, wvin_ref[...])
  scale = 1.0 / jnp.sqrt(jnp.float32(_HD))
  outs = []
  for hh in range(_HEADS):
    sl = slice(hh * _HD, (hh + 1) * _HD)
    sc = _dot_t(qm[:, sl], k[:, sl]) * scale
    key_id = lax.broadcasted_iota(jnp.int32, sc.shape, 1)
    sc = jnp.where(key_id < _N, sc, -1e30)
    m = jnp.max(sc, axis=-1, keepdims=True)
    e = jnp.exp(sc - m)
    p = e / jnp.sum(e, axis=-1, keepdims=True)
    outs.append(_dot(p, v[:, sl]))
  upd = jnp.concatenate(outs, axis=1)
  memnew_ref[...] = _ln(mem_ref[...] + upd, gm_ref[...], bm_ref[...])


def _post_body(h_ref, memnew_ref, wqt_ref, wkm_ref, wvm_ref, wo_ref, bo_ref,
               ga_ref, ba_ref, f1_ref, bf1_ref, f2_ref, bf2_ref,
               gf_ref, bf_ref, go_ref, bout_ref,
               w2sa_ref, w2sb_ref, w2na_ref, w2nb_ref, b2_ref,
               n2t_ref, selfp_ref):
  h = h_ref[...]
  mn = memnew_ref[...]
  q = _dot(h, wqt_ref[...])
  km = _dot(mn, wkm_ref[...])
  vm = _dot(mn, wvm_ref[...])
  scale = 1.0 / jnp.sqrt(jnp.float32(_HD))
  outs = []
  for hh in range(_HEADS):
    sl = slice(hh * _HD, (hh + 1) * _HD)
    sc = _dot_t(q[:, sl], km[:, sl]) * scale
    m = jnp.max(sc, axis=-1, keepdims=True)
    e = jnp.exp(sc - m)
    p = e / jnp.sum(e, axis=-1, keepdims=True)
    outs.append(_dot(p, vm[:, sl]))
  attn = _dot(jnp.concatenate(outs, axis=1), wo_ref[...]) + bo_ref[...]
  h1 = _ln(h + attn, ga_ref[...], ba_ref[...])
  ffn = _dot(jnp.maximum(_dot(h1, f1_ref[...]) + bf1_ref[...], 0.0),
             f2_ref[...]) + bf2_ref[...]
  xgw = _ln(h1 + ffn, gf_ref[...], bf_ref[...])
  hln = _ln(h, go_ref[...], bout_ref[...])
  n2 = _dot(hln, w2na_ref[...]) + _dot(xgw, w2nb_ref[...])
  n2t_ref[...] = jnp.concatenate([n2, jnp.zeros_like(n2)], axis=1)
  selfp_ref[...] = (_dot(hln, w2sa_ref[...]) + _dot(xgw, w2sb_ref[...])
                    + b2_ref[...])


def _final_body(selfp_ref, p2_ref, deg_ref, out_ref):
  n2 = (p2_ref[0, :, :_D_OUT] + p2_ref[1, :, :_D_OUT]) / deg_ref[...][:, None]
  logits = selfp_ref[...] + n2
  m = jnp.max(logits, axis=-1, keepdims=True)
  lse = jnp.log(jnp.sum(jnp.exp(logits - m), axis=-1, keepdims=True))
  out_ref[...] = logits - m - lse


def kernel(x, edge_index, W1_self, W1_neigh, b1, memory, Wq_m, Wk_in, Wv_in,
           Wq_t, Wk_m, Wv_m, Wo, bo, g_mem, b_mem, g_attn, b_attn,
           fc1, bfc1, fc2, bfc2, g_ffn, b_ffn, g_out, b_out,
           W2_self, W2_neigh, b2):
  x_pad = jnp.pad(x, ((0, _NPAD - _N), (0, 0)))
  src = edge_index[0].astype(jnp.int32)
  dst = edge_index[1].astype(jnp.int32)
  zeros_chunk = jnp.zeros((_CHUNK, _D_H), jnp.float32)

  table1, xs = pl.pallas_call(
      _pre_body,
      grid=(_GRID,),
      in_specs=[
          pl.BlockSpec((_BLK, _D_IN), lambda i: (i, 0)),
          pl.BlockSpec((_D_IN, _D_H), lambda i: (0, 0)),
          pl.BlockSpec((_D_IN, _D_H), lambda i: (0, 0)),
          pl.BlockSpec((_D_H,), lambda i: (0,)),
      ],
      out_specs=[
          pl.BlockSpec((_BLK, _D_H), lambda i: (i, 0)),
          pl.BlockSpec((_BLK, _D_H), lambda i: (i, 0)),
      ],
      out_shape=[
          jax.ShapeDtypeStruct((_NPAD, _D_H), jnp.float32),
          jax.ShapeDtypeStruct((_NPAD, _D_H), jnp.float32),
      ],
  )(x_pad, W1_neigh, W1_self, b1)

  partials1, deg_parts = _make_edge_agg(True)(table1, src, dst, zeros_chunk)

  h, deg, mem_new = pl.pallas_call(
      _mid_body,
      out_shape=[
          jax.ShapeDtypeStruct((_NPAD, _D_H), jnp.float32),
          jax.ShapeDtypeStruct((_NPAD,), jnp.float32),
          jax.ShapeDtypeStruct((8, _D_H), jnp.float32),
      ],
  )(xs, partials1, deg_parts, memory, Wq_m, Wk_in, Wv_in, g_mem, b_mem)

  n2table, selfpart = pl.pallas_call(
      _post_body,
      grid=(_GRID,),
      in_specs=[
          pl.BlockSpec((_BLK, _D_H), lambda i: (i, 0)),
          pl.BlockSpec((8, _D_H), lambda i: (0, 0)),
          pl.BlockSpec((_D_H, _D_H), lambda i: (0, 0)),
          pl.BlockSpec((_D_H, _D_H), lambda i: (0, 0)),
          pl.BlockSpec((_D_H, _D_H), lambda i: (0, 0)),
          pl.BlockSpec((_D_H, _D_H), lambda i: (0, 0)),
          pl.BlockSpec((_D_H,), lambda i: (0,)),
          pl.BlockSpec((_D_H,), lambda i: (0,)),
          pl.BlockSpec((_D_H,), lambda i: (0,)),
          pl.BlockSpec((_D_H, 256), lambda i: (0, 0)),
          pl.BlockSpec((256,), lambda i: (0,)),
          pl.BlockSpec((256, _D_H), lambda i: (0, 0)),
          pl.BlockSpec((_D_H,), lambda i: (0,)),
          pl.BlockSpec((_D_H,), lambda i: (0,)),
          pl.BlockSpec((_D_H,), lambda i: (0,)),
          pl.BlockSpec((_D_H,), lambda i: (0,)),
          pl.BlockSpec((_D_H,), lambda i: (0,)),
          pl.BlockSpec((_D_H, _D_OUT), lambda i: (0, 0)),
          pl.BlockSpec((_D_H, _D_OUT), lambda i: (0, 0)),
          pl.BlockSpec((_D_H, _D_OUT), lambda i: (0, 0)),
          pl.BlockSpec((_D_H, _D_OUT), lambda i: (0, 0)),
          pl.BlockSpec((_D_OUT,), lambda i: (0,)),
      ],
      out_specs=[
          pl.BlockSpec((_BLK, _D_H), lambda i: (i, 0)),
          pl.BlockSpec((_BLK, _D_OUT), lambda i: (i, 0)),
      ],
      out_shape=[
          jax.ShapeDtypeStruct((_NPAD, _D_H), jnp.float32),
          jax.ShapeDtypeStruct((_NPAD, _D_OUT), jnp.float32),
      ],
  )(h, mem_new, Wq_t, Wk_m, Wv_m, Wo, bo, g_attn, b_attn,
    fc1, bfc1, fc2, bfc2, g_ffn, b_ffn, g_out, b_out,
    W2_self[:_D_H], W2_self[_D_H:], W2_neigh[:_D_H], W2_neigh[_D_H:], b2)

  partials2 = _make_edge_agg(False)(n2table, src, dst, zeros_chunk)

  out = pl.pallas_call(
      _final_body,
      grid=(_GRID,),
      in_specs=[
          pl.BlockSpec((_BLK, _D_OUT), lambda i: (i, 0)),
          pl.BlockSpec((_NCORES, _BLK, _D_H), lambda i: (0, i, 0)),
          pl.BlockSpec((_BLK,), lambda i: (i,)),
      ],
      out_specs=pl.BlockSpec((_BLK, _D_OUT), lambda i: (i, 0)),
      out_shape=jax.ShapeDtypeStruct((_NPAD, _D_OUT), jnp.float32),
  )(selfpart, partials2, deg)

  return out[:_N]
